# baseline (speedup 1.0000x reference)
import jax
import jax.numpy as jnp
from jax.experimental import pallas as pl
from jax.experimental.pallas import tpu as pltpu

N_MOD = 2
SUB = 2048
N = N_MOD * SUB
DIM = 32
K = 20
ROWS = 512
TILES_PER_BLOCK = SUB // ROWS
GRID = N_MOD * TILES_PER_BLOCK

def _oems_pairs(n):
    pairs = []
    p = 1
    while p < n:
        k = p
        while k >= 1:
            for j in range(k % p, n - k, 2 * k):
                for i in range(min(k, n - j - k)):
                    if (i + j) // (p * 2) == (i + j + k) // (p * 2):
                        pairs.append((i + j, i + j + k))
            k //= 2
        p *= 2
    return pairs


_SORT16 = _oems_pairs(16)


def _merge_top(aa, bb, keep, only=None):
    la, lb = len(aa), len(bb)
    t = []
    for i in range(32):
        a = aa[i] if i < la else None
        b = bb[31 - i] if 31 - i < lb else None
        if a is None:
            t.append(b)
        elif b is None:
            t.append(a)
        else:
            t.append(jnp.maximum(a, b))
    cur = {only} if only is not None else set(range(keep))
    marks = []
    for k in (1, 2, 4, 8, 16):
        marks.append((k, set(cur)))
        nxt = set()
        for i in range(32):
            if (i & k) == 0 and (i in cur or (i + k) in cur):
                nxt.add(i)
                nxt.add(i + k)
        cur = nxt
    marks.reverse()
    for (k, need) in marks:
        for i in range(32):
            if (i & k) == 0:
                ni, nk = i in need, (i + k) in need
                if not (ni or nk):
                    continue
                a, b = t[i], t[i + k]
                if ni and nk:
                    t[i], t[i + k] = jnp.maximum(a, b), jnp.minimum(a, b)
                elif ni:
                    t[i] = jnp.maximum(a, b)
                else:
                    t[i + k] = jnp.minimum(a, b)
    if only is not None:
        return t[only]
    return t[:keep]


def _tile_kernel(e1, e2, wt, out_ref):
    f1a = jnp.dot(e1[0], wt[0], preferred_element_type=jnp.float32)
    f1b = jnp.dot(e1[1], wt[1], preferred_element_type=jnp.float32)
    top = jax.lax.dot_general(e2[0], f1a, (((1,), (1,)), ((), ())),
                              preferred_element_type=jnp.float32)
    bot = jax.lax.dot_general(e2[1], f1b, (((1,), (1,)), ((), ())),
                              preferred_element_type=jnp.float32)
    tile_t = jnp.concatenate([top, bot], axis=0)
    tile_n = tile_t.T

    v = tile_t.reshape(32, 16, 8, ROWS)
    cols = [v[:, c] for c in range(16)]
    for (i, j) in _SORT16:
        a, b = cols[i], cols[j]
        cols[i], cols[j] = jnp.maximum(a, b), jnp.minimum(a, b)
    m = _merge_top([cols[c][:16] for c in range(16)],
                   [cols[c][16:] for c in range(16)], K)
    while m[0].shape[0] > 1:
        h = m[0].shape[0] // 2
        m = _merge_top([x[:h] for x in m], [x[h:] for x in m], K)
    m = [x[0] for x in m]
    while m[0].shape[0] > 2:
        h = m[0].shape[0] // 2
        m = _merge_top([x[:h] for x in m], [x[h:] for x in m], K)
    thresh = _merge_top([x[:1] for x in m], [x[1:] for x in m], K,
                        only=K - 1)
    thresh_col = thresh.T

    out_ref[...] = jnp.where(tile_n >= thresh_col, tile_n, 0.0)


@jax.jit
def kernel(idx, emb1_w, emb2_w, lin1_w, lin1_b, lin2_w, lin2_b):
    del idx
    del lin1_b, lin2_b
    wt = jnp.einsum("mji,mjk->mik", lin1_w, lin2_w,
                    preferred_element_type=jnp.float32)
    grid_spec = pl.GridSpec(
        grid=(GRID,),
        in_specs=[
            pl.BlockSpec((2, ROWS, DIM),
                         lambda g: (g // TILES_PER_BLOCK, g % TILES_PER_BLOCK, 0)),
            pl.BlockSpec((2, SUB, DIM), lambda g: (g // TILES_PER_BLOCK, 0, 0)),
            pl.BlockSpec((2, DIM, DIM), lambda g: (g // TILES_PER_BLOCK, 0, 0)),
        ],
        out_specs=pl.BlockSpec((ROWS, N), lambda g: (g, 0)),
    )
    return pl.pallas_call(
        _tile_kernel,
        grid_spec=grid_spec,
        out_shape=jax.ShapeDtypeStruct((N, N), jnp.float32),
        compiler_params=pltpu.CompilerParams(
            dimension_semantics=("parallel",),
        ),
    )(emb1_w, emb2_w, wt)


# speedup vs baseline: 1.0146x; 1.0146x over previous
import jax
import jax.numpy as jnp
from jax.experimental import pallas as pl
from jax.experimental.pallas import tpu as pltpu

N_MOD = 2
SUB = 2048
N = N_MOD * SUB
DIM = 32
K = 20
ROWS = 512
TILES_PER_BLOCK = SUB // ROWS
GRID = N_MOD * TILES_PER_BLOCK

def _oems_pairs(n):
    pairs = []
    p = 1
    while p < n:
        k = p
        while k >= 1:
            for j in range(k % p, n - k, 2 * k):
                for i in range(min(k, n - j - k)):
                    if (i + j) // (p * 2) == (i + j + k) // (p * 2):
                        pairs.append((i + j, i + j + k))
            k //= 2
        p *= 2
    return pairs


_SORT16 = _oems_pairs(16)


def _merge_top(aa, bb, keep, only=None):
    la, lb = len(aa), len(bb)
    t = []
    for i in range(32):
        a = aa[i] if i < la else None
        b = bb[31 - i] if 31 - i < lb else None
        if a is None:
            t.append(b)
        elif b is None:
            t.append(a)
        else:
            t.append(jnp.maximum(a, b))
    cur = {only} if only is not None else set(range(keep))
    marks = []
    for k in (1, 2, 4, 8, 16):
        marks.append((k, set(cur)))
        nxt = set()
        for i in range(32):
            if (i & k) == 0 and (i in cur or (i + k) in cur):
                nxt.add(i)
                nxt.add(i + k)
        cur = nxt
    marks.reverse()
    for (k, need) in marks:
        for i in range(32):
            if (i & k) == 0:
                ni, nk = i in need, (i + k) in need
                if not (ni or nk):
                    continue
                a, b = t[i], t[i + k]
                if ni and nk:
                    t[i], t[i + k] = jnp.maximum(a, b), jnp.minimum(a, b)
                elif ni:
                    t[i] = jnp.maximum(a, b)
                else:
                    t[i + k] = jnp.minimum(a, b)
    if only is not None:
        return t[only]
    return t[:keep]


def _tile_kernel(e1, e2, l1w, l1b, l2w, l2b, out_ref):
    nv1a = jnp.dot(e1[0], l1w[0].T, preferred_element_type=jnp.float32) + l1b[0]
    nv1b = jnp.dot(e1[1], l1w[1].T, preferred_element_type=jnp.float32) + l1b[1]
    nv2a = jnp.dot(e2[0], l2w[0].T, preferred_element_type=jnp.float32) + l2b[0]
    nv2b = jnp.dot(e2[1], l2w[1].T, preferred_element_type=jnp.float32) + l2b[1]
    top = jax.lax.dot_general(nv2a, nv1a, (((1,), (1,)), ((), ())),
                              preferred_element_type=jnp.float32)
    bot = jax.lax.dot_general(nv2b, nv1b, (((1,), (1,)), ((), ())),
                              preferred_element_type=jnp.float32)
    tile_t = jnp.concatenate([top, bot], axis=0)
    tile_n = tile_t.T

    v = tile_t.reshape(32, 16, 8, ROWS)
    cols = [v[:, c] for c in range(16)]
    for (i, j) in _SORT16:
        a, b = cols[i], cols[j]
        cols[i], cols[j] = jnp.maximum(a, b), jnp.minimum(a, b)
    m = _merge_top([cols[c][:16] for c in range(16)],
                   [cols[c][16:] for c in range(16)], K)
    while m[0].shape[0] > 1:
        h = m[0].shape[0] // 2
        m = _merge_top([x[:h] for x in m], [x[h:] for x in m], K)
    m = [x[0] for x in m]
    while m[0].shape[0] > 2:
        h = m[0].shape[0] // 2
        m = _merge_top([x[:h] for x in m], [x[h:] for x in m], K)
    thresh = _merge_top([x[:1] for x in m], [x[1:] for x in m], K,
                        only=K - 1)
    thresh_col = thresh.T

    out_ref[...] = jnp.where(tile_n >= thresh_col, tile_n, 0.0)


@jax.jit
def kernel(idx, emb1_w, emb2_w, lin1_w, lin1_b, lin2_w, lin2_b):
    del idx
    lin1_b3 = lin1_b[:, None, :]
    lin2_b3 = lin2_b[:, None, :]
    grid_spec = pl.GridSpec(
        grid=(GRID,),
        in_specs=[
            pl.BlockSpec((2, ROWS, DIM),
                         lambda g: (g // TILES_PER_BLOCK, g % TILES_PER_BLOCK, 0)),
            pl.BlockSpec((2, SUB, DIM), lambda g: (g // TILES_PER_BLOCK, 0, 0)),
            pl.BlockSpec((2, DIM, DIM), lambda g: (g // TILES_PER_BLOCK, 0, 0)),
            pl.BlockSpec((2, 1, DIM), lambda g: (g // TILES_PER_BLOCK, 0, 0)),
            pl.BlockSpec((2, DIM, DIM), lambda g: (g // TILES_PER_BLOCK, 0, 0)),
            pl.BlockSpec((2, 1, DIM), lambda g: (g // TILES_PER_BLOCK, 0, 0)),
        ],
        out_specs=pl.BlockSpec((ROWS, N), lambda g: (g, 0)),
    )
    return pl.pallas_call(
        _tile_kernel,
        grid_spec=grid_spec,
        out_shape=jax.ShapeDtypeStruct((N, N), jnp.float32),
        compiler_params=pltpu.CompilerParams(
            dimension_semantics=("parallel",),
        ),
    )(emb1_w, emb2_w, lin1_w, lin1_b3, lin2_w, lin2_b3)
